# baseline (speedup 1.0000x reference)
import functools
import jax
import jax.numpy as jnp
from jax import lax
from jax.experimental import pallas as pl
from jax.experimental.pallas import tpu as pltpu
from jax.experimental.pallas import tpu_sc as plsc

D_MODEL = 64
NUM_WORKERS = 32
CHUNK = 128


def _make_gather(n_total: int):
    per_w = n_total // NUM_WORKERS
    n_chunks = per_w // CHUNK
    mesh = plsc.VectorSubcoreMesh(core_axis_name="c", subcore_axis_name="s")

    @functools.partial(
        pl.kernel,
        mesh=mesh,
        out_type=jax.ShapeDtypeStruct((n_total, D_MODEL), jnp.float32),
        scratch_types=[
            pltpu.VMEM((per_w,), jnp.int32),
            pltpu.VMEM((CHUNK, D_MODEL), jnp.float32),
            pltpu.SemaphoreType.DMA,
        ],
        compiler_params=pltpu.CompilerParams(use_tc_tiling_on_sc=False),
    )
    def gather_kernel(idx_hbm, table_hbm, out_hbm, idx_v, rows_v, sem):
        wid = lax.axis_index("s") * 2 + lax.axis_index("c")
        base = wid * per_w
        pltpu.sync_copy(idx_hbm.at[pl.ds(base, per_w)], idx_v)

        def step(i, carry):
            off = i * CHUNK
            pltpu.async_copy(
                table_hbm.at[idx_v.at[pl.ds(off, CHUNK)]], rows_v, sem
            ).wait()
            pltpu.sync_copy(rows_v, out_hbm.at[pl.ds(base + off, CHUNK)])
            return carry

        lax.fori_loop(0, n_chunks, step, 0)

    return gather_kernel


def kernel(input, table):
    b, s = input.shape
    idx_flat = input.reshape(-1).astype(jnp.int32)
    out = _make_gather(b * s)(idx_flat, table)
    return out.reshape(b, s, D_MODEL)


# speedup vs baseline: 1.0436x; 1.0436x over previous
import functools
import jax
import jax.numpy as jnp
from jax import lax
from jax.experimental import pallas as pl
from jax.experimental.pallas import tpu as pltpu
from jax.experimental.pallas import tpu_sc as plsc

D_MODEL = 64
NUM_WORKERS = 32
CHUNK = 128
NBUF = 10


def _make_gather(n_total: int):
    per_w = n_total // NUM_WORKERS
    n_chunks = per_w // CHUNK
    n_super = n_chunks // NBUF
    mesh = plsc.VectorSubcoreMesh(core_axis_name="c", subcore_axis_name="s")

    @functools.partial(
        pl.kernel,
        mesh=mesh,
        out_type=jax.ShapeDtypeStruct((n_total, D_MODEL), jnp.float32),
        scratch_types=[
            pltpu.VMEM((per_w,), jnp.int32),
            pltpu.VMEM((NBUF, CHUNK, D_MODEL), jnp.float32),
            pltpu.SemaphoreType.DMA((NBUF,)),
            pltpu.SemaphoreType.DMA((NBUF,)),
        ],
        compiler_params=pltpu.CompilerParams(use_tc_tiling_on_sc=False),
    )
    def gather_kernel(idx_hbm, table_hbm, out_hbm, idx_v, rows_v, sem_g, sem_w):
        wid = lax.axis_index("s") * 2 + lax.axis_index("c")
        base = wid * per_w
        pltpu.sync_copy(idx_hbm.at[pl.ds(base, per_w)], idx_v)

        def fire_g(chunk_off, b):
            pltpu.async_copy(
                table_hbm.at[idx_v.at[pl.ds(chunk_off * CHUNK, CHUNK)]],
                rows_v.at[b],
                sem_g.at[b],
            )

        def wait_g(b):
            pltpu.make_async_copy(
                table_hbm.at[idx_v.at[pl.ds(0, CHUNK)]],
                rows_v.at[b],
                sem_g.at[b],
            ).wait()

        def fire_w(chunk_off, b):
            pltpu.async_copy(
                rows_v.at[b],
                out_hbm.at[pl.ds(base + chunk_off * CHUNK, CHUNK)],
                sem_w.at[b],
            )

        def wait_w(b):
            pltpu.make_async_copy(
                rows_v.at[b],
                out_hbm.at[pl.ds(base, CHUNK)],
                sem_w.at[b],
            ).wait()

        for b in range(NBUF):
            fire_g(b, b)

        def step(j, carry):
            for b in range(NBUF):
                wait_g(b)
                fire_w(j * NBUF + b, b)
            for b in range(NBUF):
                wait_w(b)
                fire_g((j + 1) * NBUF + b, b)
            return carry

        lax.fori_loop(0, n_super - 1, step, 0)

        for b in range(NBUF):
            wait_g(b)
            fire_w((n_super - 1) * NBUF + b, b)
        for b in range(NBUF):
            wait_w(b)

    return gather_kernel


def kernel(input, table):
    b, s = input.shape
    idx_flat = input.reshape(-1).astype(jnp.int32)
    out = _make_gather(b * s)(idx_flat, table)
    return out.reshape(b, s, D_MODEL)
